# baseline (speedup 1.0000x reference)
import functools

import jax
import jax.numpy as jnp
from jax import lax
from jax.experimental import pallas as pl
from jax.experimental.pallas import tpu as pltpu
from jax.experimental.pallas import tpu_sc as plsc

H = 128
NN = 10000
NE = 320000
NRES = 1000
NC, NS = 2, 16
NW = NC * NS
NPAD = 10240
EPAD = NW * NPAD
EPW = EPAD // NW
CHUNK = 64
IROWS = EPAD // CHUNK
RPW = IROWS // NW
GCH = 32
NGRP = RPW // GCH
RPAD = 1024
NODE_PAD2 = 12288


def _fold_body(eW2_ref, mW1_ref, eb2_ref, w2f_ref, bf_ref):
    for l in range(3):
        B = mW1_ref[l, H:, :]
        w2f_ref[l, :, :] = jnp.dot(eW2_ref[l], B, preferred_element_type=jnp.float32)
        bf_ref[l:l + 1, :] = jnp.dot(eb2_ref[l:l + 1, :], B,
                                     preferred_element_type=jnp.float32)


def _fold_weights(eW2, mW1, eb2):
    return pl.pallas_call(
        _fold_body,
        out_shape=[jax.ShapeDtypeStruct((3, 64, H), jnp.float32),
                   jax.ShapeDtypeStruct((3, H), jnp.float32)],
    )(eW2, mW1, eb2)


def _enc_body(x_ref, wenc_ref, benc_ref, a0_ref, mb10_ref, h_ref, hp0_ref):
    h = jax.nn.relu(jnp.dot(x_ref[...], wenc_ref[...],
                            preferred_element_type=jnp.float32) + benc_ref[...])
    h_ref[...] = h
    hp0_ref[...] = jnp.dot(h, a0_ref[...],
                           preferred_element_type=jnp.float32) + mb10_ref[...]


def _encode(x_pad, W_enc, b_enc2, A0, mb10):
    br = 2048
    grid = NPAD // br
    return pl.pallas_call(
        _enc_body,
        grid=(grid,),
        in_specs=[
            pl.BlockSpec((br, H), lambda i: (i, 0)),
            pl.BlockSpec((H, H), lambda i: (0, 0)),
            pl.BlockSpec((1, H), lambda i: (0, 0)),
            pl.BlockSpec((H, H), lambda i: (0, 0)),
            pl.BlockSpec((1, H), lambda i: (0, 0)),
        ],
        out_specs=[pl.BlockSpec((br, H), lambda i: (i, 0)),
                   pl.BlockSpec((br, H), lambda i: (i, 0))],
        out_shape=[jax.ShapeDtypeStruct((NPAD, H), jnp.float32),
                   jax.ShapeDtypeStruct((NPAD, H), jnp.float32)],
    )(x_pad, W_enc, b_enc2, A0, mb10)


def _edge_mlp_body(ea_ref, eW1_ref, eb1_ref, w2f_ref, bf_ref, ep_ref, *, l):
    ea = ea_ref[...]
    a = jax.nn.relu(jnp.dot(ea, eW1_ref[l],
                            preferred_element_type=jnp.float32)
                    + eb1_ref[l, :][None, :])
    ep_ref[...] = (jnp.dot(a, w2f_ref[l],
                           preferred_element_type=jnp.float32)
                   + bf_ref[l, :][None, :])


def _edge_mlp_layer(l, ea_pad, eW1, eb1, W2f, bf):
    be = 4096
    grid = EPAD // be
    return pl.pallas_call(
        functools.partial(_edge_mlp_body, l=l),
        grid=(grid,),
        in_specs=[
            pl.BlockSpec((be, 16), lambda i: (i, 0)),
            pl.BlockSpec((3, 16, 64), lambda i: (0, 0, 0)),
            pl.BlockSpec((3, 64), lambda i: (0, 0)),
            pl.BlockSpec((3, 64, H), lambda i: (0, 0, 0)),
            pl.BlockSpec((3, H), lambda i: (0, 0)),
        ],
        out_specs=pl.BlockSpec((be, H), lambda i: (i, 0)),
        out_shape=jax.ShapeDtypeStruct((EPAD, H), jnp.float32),
    )(ea_pad, eW1, eb1, W2f, bf)


def _tail_body(s0_ref, s1_ref, c0_ref, c1_ref, h_ref, mW2_ref, mb2_ref,
               gWih_ref, gWhh_ref, gbih_ref, gbhh_ref, lng_ref, lnb_ref,
               an_ref, mb1n_ref, h_out_ref, hp_out_ref, *, with_next):
    cnt = c0_ref[...] + c1_ref[...]
    s = s0_ref[...] + s1_ref[...]
    summed = (jnp.dot(s, mW2_ref[...], preferred_element_type=jnp.float32)
              + cnt * mb2_ref[...])
    aggr = summed / jnp.maximum(cnt, 1.0)
    h = h_ref[...]
    gi = lax.dot_general(aggr, gWih_ref[...], (((1,), (1,)), ((), ())),
                         preferred_element_type=jnp.float32) + gbih_ref[...]
    gh = lax.dot_general(h, gWhh_ref[...], (((1,), (1,)), ((), ())),
                         preferred_element_type=jnp.float32) + gbhh_ref[...]
    r = jax.nn.sigmoid(gi[:, :H] + gh[:, :H])
    z = jax.nn.sigmoid(gi[:, H:2 * H] + gh[:, H:2 * H])
    n = jnp.tanh(gi[:, 2 * H:] + r * gh[:, 2 * H:])
    h_new = (1.0 - z) * n + z * h
    mu = jnp.mean(h_new, axis=-1, keepdims=True)
    var = jnp.mean(jnp.square(h_new - mu), axis=-1, keepdims=True)
    h_next = (h_new - mu) / jnp.sqrt(var + 1e-5) * lng_ref[...] + lnb_ref[...]
    h_out_ref[...] = h_next
    if with_next:
        hp_out_ref[...] = (jnp.dot(h_next, an_ref[...],
                                   preferred_element_type=jnp.float32)
                           + mb1n_ref[...])
    else:
        hp_out_ref[...] = h_next


def _layer_tail(S, C, h, mW2l, mb2l, gWihl, gWhhl, gbihl, gbhhl, lngl, lnbl,
                A_next, mb1_next, with_next):
    br = 2048
    grid = NPAD // br
    full = lambda shape: pl.BlockSpec(shape, lambda i: tuple(0 for _ in shape))
    blk = pl.BlockSpec((br, H), lambda i: (i, 0))
    col = pl.BlockSpec((br, 1), lambda i: (i, 0))
    s0 = S[0]
    s1 = S[1]
    c0 = C[0].reshape(NPAD, 1)
    c1 = C[1].reshape(NPAD, 1)
    return pl.pallas_call(
        functools.partial(_tail_body, with_next=with_next),
        grid=(grid,),
        in_specs=[blk, blk, col, col, blk,
                  full((H, H)), full((1, H)),
                  full((3 * H, H)), full((3 * H, H)),
                  full((1, 3 * H)), full((1, 3 * H)),
                  full((1, H)), full((1, H)),
                  full((H, H)), full((1, H))],
        out_specs=[blk, blk],
        out_shape=[jax.ShapeDtypeStruct((NPAD, H), jnp.float32),
                   jax.ShapeDtypeStruct((NPAD, H), jnp.float32)],
    )(s0, s1, c0, c1, h, mW2l, mb2l, gWihl, gWhhl, gbihl, gbhhl, lngl, lnbl,
      A_next, mb1_next)


def _head_body(rs0_ref, rs1_ref, rc0_ref, rc1_ref, hW1_ref, hb1_ref,
               hW2_ref, hb2_ref, out_ref):
    rc = rc0_ref[...] + rc1_ref[...]
    rx = (rs0_ref[...] + rs1_ref[...]) / jnp.maximum(rc, 1.0)
    a = jax.nn.relu(jnp.dot(rx, hW1_ref[...],
                            preferred_element_type=jnp.float32) + hb1_ref[...])
    out_ref[...] = jnp.dot(a, hW2_ref[...],
                           preferred_element_type=jnp.float32) + hb2_ref[...]


def _head(RS, RC, hW1, hb12, hW2, hb22):
    rs0, rs1 = RS[0], RS[1]
    rc0 = RC[0].reshape(RPAD, 1)
    rc1 = RC[1].reshape(RPAD, 1)
    return pl.pallas_call(
        _head_body,
        out_shape=jax.ShapeDtypeStruct((RPAD, 1), jnp.float32),
    )(rs0, rs1, rc0, rc1, hW1, hb12, hW2, hb22)


def _sc_edge_body(hp_hbm, ep_hbm, src_hbm, dst_hbm, s_hbm, c_hbm,
                  acc, accc, src_b, dst_b, gath0, gath1, epb0, epb1,
                  ones_b, zc, es0, es1, gs0, gs1):
    cid = lax.axis_index("c")
    sid = lax.axis_index("s")
    wid = cid * NS + sid
    gaths = (gath0, gath1)
    epbs = (epb0, epb1)
    esems = (es0, es1)
    gsems = (gs0, gs1)

    def zrow(i, carry):
        for j in range(8):
            gath0[i, pl.ds(j * 16, 16)] = jnp.zeros((16,), jnp.float32)
        return carry
    lax.fori_loop(0, CHUNK, zrow, 0)

    def zc_loop(i, carry):
        zc[pl.ds(i * 16, 16)] = jnp.zeros((16,), jnp.float32)
        return carry
    lax.fori_loop(0, 40, zc_loop, 0)

    def ones_loop(i, carry):
        ones_b[pl.ds(i * 16, 16)] = jnp.ones((16,), jnp.float32)
        return carry
    lax.fori_loop(0, CHUNK // 16, ones_loop, 0)

    zone = NPAD // NS
    for j in range(zone // CHUNK):
        pltpu.sync_copy(gath0, acc.at[pl.ds(sid * zone + j * CHUNK, CHUNK)])
    pltpu.sync_copy(zc, accc.at[pl.ds(sid * zone, zone)])
    plsc.subcore_barrier()

    def issue(g, c, b):
        ebase = (wid * RPW + g * GCH) * CHUNK + c * CHUNK
        pltpu.async_copy(ep_hbm.at[pl.ds(ebase, CHUNK)], epbs[b], esems[b])
        pltpu.async_copy(hp_hbm.at[src_b.at[c]], gaths[b], gsems[b])

    def wait_compute_scatter(c, b):
        pltpu.make_async_copy(ep_hbm.at[pl.ds(0, CHUNK)], epbs[b],
                              esems[b]).wait()
        pltpu.make_async_copy(hp_hbm.at[pl.ds(0, CHUNK)], gaths[b],
                              gsems[b]).wait()
        gath = gaths[b]
        epb = epbs[b]

        @plsc.parallel_loop(0, CHUNK, 1, unroll=4)
        def rowfn(r):
            for jj in range(8):
                sl = pl.ds(jj * 16, 16)
                gath[r, sl] = jnp.maximum(gath[r, sl] + epb[r, sl], 0.0)

        pltpu.sync_copy(gath, acc.at[dst_b.at[c]], add=True)
        pltpu.sync_copy(ones_b, accc.at[dst_b.at[c]], add=True)

    for g in range(NGRP):
        rowbase = wid * RPW + g * GCH
        pltpu.sync_copy(src_hbm.at[pl.ds(rowbase, GCH)], src_b)
        pltpu.sync_copy(dst_hbm.at[pl.ds(rowbase, GCH)], dst_b)
        issue(g, 0, 0)
        issue(g, 1, 1)

        def pair(k, carry):
            for b in range(2):
                wait_compute_scatter(2 * k + b, b)
                issue(g, 2 * k + 2 + b, b)
            return carry
        lax.fori_loop(0, GCH // 2 - 1, pair, 0)
        for b in range(2):
            wait_compute_scatter(GCH - 2 + b, b)

    plsc.subcore_barrier()
    for j in range(zone // 128):
        off = sid * zone + j * 128
        pltpu.sync_copy(acc.at[pl.ds(off, 128)],
                        s_hbm.at[cid, pl.ds(off, 128)])
    pltpu.sync_copy(accc.at[pl.ds(sid * zone, zone)], zc)
    pltpu.sync_copy(zc, c_hbm.at[pl.ds(cid * NPAD + sid * zone, zone)])


_sc_edge = pl.kernel(
    _sc_edge_body,
    out_type=[jax.ShapeDtypeStruct((NC, NPAD, H), jnp.float32),
              jax.ShapeDtypeStruct((NC * NPAD,), jnp.float32)],
    mesh=plsc.VectorSubcoreMesh(core_axis_name="c", subcore_axis_name="s"),
    scratch_types=[
        pltpu.VMEM_SHARED((NPAD, H), jnp.float32),
        pltpu.VMEM_SHARED((NPAD,), jnp.float32),
        pltpu.VMEM((GCH, CHUNK), jnp.int32),
        pltpu.VMEM((GCH, CHUNK), jnp.int32),
        pltpu.VMEM((CHUNK, H), jnp.float32),
        pltpu.VMEM((CHUNK, H), jnp.float32),
        pltpu.VMEM((CHUNK, H), jnp.float32),
        pltpu.VMEM((CHUNK, H), jnp.float32),
        pltpu.VMEM((CHUNK,), jnp.float32),
        pltpu.VMEM((NPAD // NS,), jnp.float32),
        pltpu.SemaphoreType.DMA,
        pltpu.SemaphoreType.DMA,
        pltpu.SemaphoreType.DMA,
        pltpu.SemaphoreType.DMA,
    ],
)


def _sc_res_body(h_hbm, ridx_hbm, rs_hbm, rc_hbm,
                 accr, accrc, rb, hb, ones_b, zc, sem):
    cid = lax.axis_index("c")
    sid = lax.axis_index("s")
    wid = cid * NS + sid

    def zrow(i, carry):
        for j in range(8):
            hb[i, pl.ds(j * 16, 16)] = jnp.zeros((16,), jnp.float32)
        return carry
    lax.fori_loop(0, 64, zrow, 0)

    def zc_loop(i, carry):
        zc[pl.ds(i * 16, 16)] = jnp.zeros((16,), jnp.float32)
        return carry
    lax.fori_loop(0, 4, zc_loop, 0)

    def ones_loop(i, carry):
        ones_b[pl.ds(i * 16, 16)] = jnp.ones((16,), jnp.float32)
        return carry
    lax.fori_loop(0, 8, ones_loop, 0)

    zone = RPAD // NS
    pltpu.sync_copy(hb.at[pl.ds(0, 64)], accr.at[pl.ds(sid * zone, zone)])
    pltpu.sync_copy(zc, accrc.at[pl.ds(sid * zone, zone)])
    plsc.subcore_barrier()

    rows_per_w = NODE_PAD2 // H // NW
    pltpu.sync_copy(ridx_hbm.at[wid], rb)
    for k in range(rows_per_w):
        base = (wid * rows_per_w + k) * H
        pltpu.sync_copy(h_hbm.at[pl.ds(base, H)], hb)
        pltpu.sync_copy(hb, accr.at[rb.at[k]], add=True)
        pltpu.sync_copy(ones_b, accrc.at[rb.at[k]], add=True)

    plsc.subcore_barrier()
    pltpu.sync_copy(accr.at[pl.ds(sid * zone, zone)],
                    rs_hbm.at[cid, pl.ds(sid * zone, zone)])
    pltpu.sync_copy(accrc.at[pl.ds(sid * zone, zone)], zc)
    pltpu.sync_copy(zc, rc_hbm.at[pl.ds(cid * RPAD + sid * zone, zone)])


_sc_res = pl.kernel(
    _sc_res_body,
    out_type=[jax.ShapeDtypeStruct((NC, RPAD, H), jnp.float32),
              jax.ShapeDtypeStruct((NC * RPAD,), jnp.float32)],
    mesh=plsc.VectorSubcoreMesh(core_axis_name="c", subcore_axis_name="s"),
    scratch_types=[
        pltpu.VMEM_SHARED((RPAD, H), jnp.float32),
        pltpu.VMEM_SHARED((RPAD,), jnp.float32),
        pltpu.VMEM((8, H), jnp.int32),
        pltpu.VMEM((H, H), jnp.float32),
        pltpu.VMEM((H,), jnp.float32),
        pltpu.VMEM((RPAD // NS,), jnp.float32),
        pltpu.SemaphoreType.DMA,
    ],
)


def kernel(x, edge_index, edge_attr, res_idx, W_enc, b_enc, eW1, eb1, eW2,
           eb2, mW1, mb1, mW2, mb2, gWih, gWhh, gbih, gbhh, lng, lnb, hW1,
           hb1, hW2, hb2):
    f32 = jnp.float32
    src = edge_index[0]
    dst = edge_index[1]

    npad_e = EPAD - NE
    pad_src = (jnp.arange(npad_e, dtype=jnp.int32) * 97) % NN
    pad_dst = NN + (jnp.arange(npad_e, dtype=jnp.int32) % (NPAD - NN))
    src2d = jnp.concatenate([src, pad_src]).reshape(IROWS, CHUNK)
    dst2d = jnp.concatenate([dst, pad_dst]).reshape(IROWS, CHUNK)
    ea_pad = jnp.concatenate(
        [edge_attr, jnp.zeros((npad_e, 16), f32)], axis=0)
    x_pad = jnp.concatenate([x, jnp.zeros((NPAD - NN, 128), f32)], axis=0)
    npad_r = NODE_PAD2 - NN
    pad_ridx = NRES + (jnp.arange(npad_r, dtype=jnp.int32) % (RPAD - NRES))
    ridx3d = jnp.concatenate([res_idx, pad_ridx]).reshape(NW, 3, H)
    ridx3d = jnp.pad(ridx3d, ((0, 0), (0, 5), (0, 0)),
                     constant_values=NRES)

    A = [mW1[l, :H, :] for l in range(3)]
    mb1_2 = [mb1[l].reshape(1, H) for l in range(3)]
    b_enc2 = b_enc.reshape(1, H)
    hb12 = hb1.reshape(1, 64)
    hb22 = hb2.reshape(1, 1)

    W2f, bf = _fold_weights(eW2, mW1, eb2)

    h, hp = _encode(x_pad, W_enc, b_enc2, A[0], mb1_2[0])

    for l in range(3):
        ep_l = _edge_mlp_layer(l, ea_pad, eW1, eb1, W2f, bf)
        S, C = _sc_edge(hp, ep_l, src2d, dst2d)
        C = C.reshape(NC, NPAD)
        with_next = l < 2
        an = A[l + 1] if with_next else A[0]
        mb1n = mb1_2[l + 1] if with_next else mb1_2[0]
        h, hp = _layer_tail(
            S, C, h, mW2[l], mb2[l].reshape(1, H), gWih[l], gWhh[l],
            gbih[l].reshape(1, 3 * H), gbhh[l].reshape(1, 3 * H),
            lng[l].reshape(1, H), lnb[l].reshape(1, H), an, mb1n, with_next)

    h_rp = jnp.concatenate(
        [h, jnp.zeros((NODE_PAD2 - NPAD, H), f32)], axis=0)
    RS, RC = _sc_res(h_rp, ridx3d)
    out2d = _head(RS, RC.reshape(NC, RPAD), hW1, hb12, hW2, hb22)
    return out2d[:NRES, 0]


# speedup vs baseline: 1.0153x; 1.0153x over previous
import functools

import jax
import jax.numpy as jnp
from jax import lax
from jax.experimental import pallas as pl
from jax.experimental.pallas import tpu as pltpu
from jax.experimental.pallas import tpu_sc as plsc

H = 128
NN = 10000
NE = 320000
NRES = 1000
NC, NS = 2, 16
NW = NC * NS
NPAD = 10240
EPAD = NW * NPAD
EPW = EPAD // NW
CHUNK = 64
IROWS = EPAD // CHUNK
RPW = IROWS // NW
GCH = 32
NGRP = RPW // GCH
RPAD = 1024
NODE_PAD2 = 12288


def _fold_body(eW2_ref, mW1_ref, eb2_ref, w2f_ref, bf_ref):
    for l in range(3):
        B = mW1_ref[l, H:, :]
        w2f_ref[l, :, :] = jnp.dot(eW2_ref[l], B, preferred_element_type=jnp.float32)
        bf_ref[l:l + 1, :] = jnp.dot(eb2_ref[l:l + 1, :], B,
                                     preferred_element_type=jnp.float32)


def _fold_weights(eW2, mW1, eb2):
    return pl.pallas_call(
        _fold_body,
        out_shape=[jax.ShapeDtypeStruct((3, 64, H), jnp.float32),
                   jax.ShapeDtypeStruct((3, H), jnp.float32)],
    )(eW2, mW1, eb2)


def _enc_body(x_ref, wenc_ref, benc_ref, a0_ref, mb10_ref, h_ref, hp0_ref):
    h = jax.nn.relu(jnp.dot(x_ref[...], wenc_ref[...],
                            preferred_element_type=jnp.float32) + benc_ref[...])
    h_ref[...] = h
    hp0_ref[...] = jnp.dot(h, a0_ref[...],
                           preferred_element_type=jnp.float32) + mb10_ref[...]


def _encode(x_pad, W_enc, b_enc2, A0, mb10):
    br = 2048
    grid = NPAD // br
    return pl.pallas_call(
        _enc_body,
        grid=(grid,),
        in_specs=[
            pl.BlockSpec((br, H), lambda i: (i, 0)),
            pl.BlockSpec((H, H), lambda i: (0, 0)),
            pl.BlockSpec((1, H), lambda i: (0, 0)),
            pl.BlockSpec((H, H), lambda i: (0, 0)),
            pl.BlockSpec((1, H), lambda i: (0, 0)),
        ],
        out_specs=[pl.BlockSpec((br, H), lambda i: (i, 0)),
                   pl.BlockSpec((br, H), lambda i: (i, 0))],
        out_shape=[jax.ShapeDtypeStruct((NPAD, H), jnp.float32),
                   jax.ShapeDtypeStruct((NPAD, H), jnp.float32)],
    )(x_pad, W_enc, b_enc2, A0, mb10)


def _edge_mlp_body(ea_ref, eW1_ref, eb1_ref, w2f_ref, bf_ref,
                   ep0_ref, ep1_ref, ep2_ref):
    ea = ea_ref[...]
    outs = (ep0_ref, ep1_ref, ep2_ref)
    for l in range(3):
        a = jax.nn.relu(jnp.dot(ea, eW1_ref[l],
                                preferred_element_type=jnp.float32)
                        + eb1_ref[l, :][None, :])
        outs[l][...] = (jnp.dot(a, w2f_ref[l],
                                preferred_element_type=jnp.float32)
                        + bf_ref[l, :][None, :])


def _edge_mlp(ea_pad, eW1, eb1, W2f, bf):
    be = 4096
    grid = EPAD // be
    ep_shape = jax.ShapeDtypeStruct((EPAD, H), jnp.float32)
    return pl.pallas_call(
        _edge_mlp_body,
        grid=(grid,),
        in_specs=[
            pl.BlockSpec((be, 16), lambda i: (i, 0)),
            pl.BlockSpec((3, 16, 64), lambda i: (0, 0, 0)),
            pl.BlockSpec((3, 64), lambda i: (0, 0)),
            pl.BlockSpec((3, 64, H), lambda i: (0, 0, 0)),
            pl.BlockSpec((3, H), lambda i: (0, 0)),
        ],
        out_specs=[pl.BlockSpec((be, H), lambda i: (i, 0))] * 3,
        out_shape=[ep_shape, ep_shape, ep_shape],
    )(ea_pad, eW1, eb1, W2f, bf)


def _tail_body(s0_ref, s1_ref, c0_ref, c1_ref, h_ref, mW2_ref, mb2_ref,
               gWih_ref, gWhh_ref, gbih_ref, gbhh_ref, lng_ref, lnb_ref,
               an_ref, mb1n_ref, h_out_ref, hp_out_ref, *, with_next):
    cnt = c0_ref[...] + c1_ref[...]
    s = s0_ref[...] + s1_ref[...]
    summed = (jnp.dot(s, mW2_ref[...], preferred_element_type=jnp.float32)
              + cnt * mb2_ref[...])
    aggr = summed / jnp.maximum(cnt, 1.0)
    h = h_ref[...]
    gi = lax.dot_general(aggr, gWih_ref[...], (((1,), (1,)), ((), ())),
                         preferred_element_type=jnp.float32) + gbih_ref[...]
    gh = lax.dot_general(h, gWhh_ref[...], (((1,), (1,)), ((), ())),
                         preferred_element_type=jnp.float32) + gbhh_ref[...]
    r = jax.nn.sigmoid(gi[:, :H] + gh[:, :H])
    z = jax.nn.sigmoid(gi[:, H:2 * H] + gh[:, H:2 * H])
    n = jnp.tanh(gi[:, 2 * H:] + r * gh[:, 2 * H:])
    h_new = (1.0 - z) * n + z * h
    mu = jnp.mean(h_new, axis=-1, keepdims=True)
    var = jnp.mean(jnp.square(h_new - mu), axis=-1, keepdims=True)
    h_next = (h_new - mu) / jnp.sqrt(var + 1e-5) * lng_ref[...] + lnb_ref[...]
    h_out_ref[...] = h_next
    if with_next:
        hp_out_ref[...] = (jnp.dot(h_next, an_ref[...],
                                   preferred_element_type=jnp.float32)
                           + mb1n_ref[...])
    else:
        hp_out_ref[...] = h_next


def _layer_tail(S, C, h, mW2l, mb2l, gWihl, gWhhl, gbihl, gbhhl, lngl, lnbl,
                A_next, mb1_next, with_next):
    br = 2048
    grid = NPAD // br
    full = lambda shape: pl.BlockSpec(shape, lambda i: tuple(0 for _ in shape))
    blk = pl.BlockSpec((br, H), lambda i: (i, 0))
    col = pl.BlockSpec((br, 1), lambda i: (i, 0))
    s0 = S[0]
    s1 = S[1]
    c0 = C[0].reshape(NPAD, 1)
    c1 = C[1].reshape(NPAD, 1)
    return pl.pallas_call(
        functools.partial(_tail_body, with_next=with_next),
        grid=(grid,),
        in_specs=[blk, blk, col, col, blk,
                  full((H, H)), full((1, H)),
                  full((3 * H, H)), full((3 * H, H)),
                  full((1, 3 * H)), full((1, 3 * H)),
                  full((1, H)), full((1, H)),
                  full((H, H)), full((1, H))],
        out_specs=[blk, blk],
        out_shape=[jax.ShapeDtypeStruct((NPAD, H), jnp.float32),
                   jax.ShapeDtypeStruct((NPAD, H), jnp.float32)],
    )(s0, s1, c0, c1, h, mW2l, mb2l, gWihl, gWhhl, gbihl, gbhhl, lngl, lnbl,
      A_next, mb1_next)


def _head_body(rs0_ref, rs1_ref, rc0_ref, rc1_ref, hW1_ref, hb1_ref,
               hW2_ref, hb2_ref, out_ref):
    rc = rc0_ref[...] + rc1_ref[...]
    rx = (rs0_ref[...] + rs1_ref[...]) / jnp.maximum(rc, 1.0)
    a = jax.nn.relu(jnp.dot(rx, hW1_ref[...],
                            preferred_element_type=jnp.float32) + hb1_ref[...])
    out_ref[...] = jnp.dot(a, hW2_ref[...],
                           preferred_element_type=jnp.float32) + hb2_ref[...]


def _head(RS, RC, hW1, hb12, hW2, hb22):
    rs0, rs1 = RS[0], RS[1]
    rc0 = RC[0].reshape(RPAD, 1)
    rc1 = RC[1].reshape(RPAD, 1)
    return pl.pallas_call(
        _head_body,
        out_shape=jax.ShapeDtypeStruct((RPAD, 1), jnp.float32),
    )(rs0, rs1, rc0, rc1, hW1, hb12, hW2, hb22)


def _sc_edge_body(hp_hbm, ep_hbm, src_hbm, dst_hbm, s_hbm, c_hbm,
                  acc, accc, src_b, dst_b, gath0, gath1, epb0, epb1,
                  ones_b, zc, es0, es1, gs0, gs1):
    cid = lax.axis_index("c")
    sid = lax.axis_index("s")
    wid = cid * NS + sid
    gaths = (gath0, gath1)
    epbs = (epb0, epb1)
    esems = (es0, es1)
    gsems = (gs0, gs1)

    def zrow(i, carry):
        for j in range(8):
            gath0[i, pl.ds(j * 16, 16)] = jnp.zeros((16,), jnp.float32)
        return carry
    lax.fori_loop(0, CHUNK, zrow, 0)

    def zc_loop(i, carry):
        zc[pl.ds(i * 16, 16)] = jnp.zeros((16,), jnp.float32)
        return carry
    lax.fori_loop(0, 40, zc_loop, 0)

    def ones_loop(i, carry):
        ones_b[pl.ds(i * 16, 16)] = jnp.ones((16,), jnp.float32)
        return carry
    lax.fori_loop(0, CHUNK // 16, ones_loop, 0)

    zone = NPAD // NS
    for j in range(zone // CHUNK):
        pltpu.sync_copy(gath0, acc.at[pl.ds(sid * zone + j * CHUNK, CHUNK)])
    pltpu.sync_copy(zc, accc.at[pl.ds(sid * zone, zone)])
    plsc.subcore_barrier()

    def issue(g, c, b):
        ebase = (wid * RPW + g * GCH) * CHUNK + c * CHUNK
        pltpu.async_copy(ep_hbm.at[pl.ds(ebase, CHUNK)], epbs[b], esems[b])
        pltpu.async_copy(hp_hbm.at[src_b.at[c]], gaths[b], gsems[b])

    def wait_compute_scatter(c, b):
        pltpu.make_async_copy(ep_hbm.at[pl.ds(0, CHUNK)], epbs[b],
                              esems[b]).wait()
        pltpu.make_async_copy(hp_hbm.at[pl.ds(0, CHUNK)], gaths[b],
                              gsems[b]).wait()
        gath = gaths[b]
        epb = epbs[b]

        @plsc.parallel_loop(0, CHUNK, 1, unroll=4)
        def rowfn(r):
            for jj in range(8):
                sl = pl.ds(jj * 16, 16)
                gath[r, sl] = jnp.maximum(gath[r, sl] + epb[r, sl], 0.0)

        pltpu.sync_copy(gath, acc.at[dst_b.at[c]], add=True)
        pltpu.sync_copy(ones_b, accc.at[dst_b.at[c]], add=True)

    for g in range(NGRP):
        rowbase = wid * RPW + g * GCH
        pltpu.sync_copy(src_hbm.at[pl.ds(rowbase, GCH)], src_b)
        pltpu.sync_copy(dst_hbm.at[pl.ds(rowbase, GCH)], dst_b)
        issue(g, 0, 0)
        issue(g, 1, 1)

        def pair(k, carry):
            for b in range(2):
                wait_compute_scatter(2 * k + b, b)
                issue(g, 2 * k + 2 + b, b)
            return carry
        lax.fori_loop(0, GCH // 2 - 1, pair, 0)
        for b in range(2):
            wait_compute_scatter(GCH - 2 + b, b)

    plsc.subcore_barrier()
    for j in range(zone // 128):
        off = sid * zone + j * 128
        pltpu.sync_copy(acc.at[pl.ds(off, 128)],
                        s_hbm.at[cid, pl.ds(off, 128)])
    pltpu.sync_copy(accc.at[pl.ds(sid * zone, zone)], zc)
    pltpu.sync_copy(zc, c_hbm.at[pl.ds(cid * NPAD + sid * zone, zone)])


_sc_edge = pl.kernel(
    _sc_edge_body,
    out_type=[jax.ShapeDtypeStruct((NC, NPAD, H), jnp.float32),
              jax.ShapeDtypeStruct((NC * NPAD,), jnp.float32)],
    mesh=plsc.VectorSubcoreMesh(core_axis_name="c", subcore_axis_name="s"),
    scratch_types=[
        pltpu.VMEM_SHARED((NPAD, H), jnp.float32),
        pltpu.VMEM_SHARED((NPAD,), jnp.float32),
        pltpu.VMEM((GCH, CHUNK), jnp.int32),
        pltpu.VMEM((GCH, CHUNK), jnp.int32),
        pltpu.VMEM((CHUNK, H), jnp.float32),
        pltpu.VMEM((CHUNK, H), jnp.float32),
        pltpu.VMEM((CHUNK, H), jnp.float32),
        pltpu.VMEM((CHUNK, H), jnp.float32),
        pltpu.VMEM((CHUNK,), jnp.float32),
        pltpu.VMEM((NPAD // NS,), jnp.float32),
        pltpu.SemaphoreType.DMA,
        pltpu.SemaphoreType.DMA,
        pltpu.SemaphoreType.DMA,
        pltpu.SemaphoreType.DMA,
    ],
)


def _sc_res_body(h_hbm, ridx_hbm, rs_hbm, rc_hbm,
                 accr, accrc, rb, hb, ones_b, zc, sem):
    cid = lax.axis_index("c")
    sid = lax.axis_index("s")
    wid = cid * NS + sid

    def zrow(i, carry):
        for j in range(8):
            hb[i, pl.ds(j * 16, 16)] = jnp.zeros((16,), jnp.float32)
        return carry
    lax.fori_loop(0, 64, zrow, 0)

    def zc_loop(i, carry):
        zc[pl.ds(i * 16, 16)] = jnp.zeros((16,), jnp.float32)
        return carry
    lax.fori_loop(0, 4, zc_loop, 0)

    def ones_loop(i, carry):
        ones_b[pl.ds(i * 16, 16)] = jnp.ones((16,), jnp.float32)
        return carry
    lax.fori_loop(0, 8, ones_loop, 0)

    zone = RPAD // NS
    pltpu.sync_copy(hb.at[pl.ds(0, 64)], accr.at[pl.ds(sid * zone, zone)])
    pltpu.sync_copy(zc, accrc.at[pl.ds(sid * zone, zone)])
    plsc.subcore_barrier()

    rows_per_w = NODE_PAD2 // H // NW
    pltpu.sync_copy(ridx_hbm.at[wid], rb)
    for k in range(rows_per_w):
        base = (wid * rows_per_w + k) * H
        pltpu.sync_copy(h_hbm.at[pl.ds(base, H)], hb)
        pltpu.sync_copy(hb, accr.at[rb.at[k]], add=True)
        pltpu.sync_copy(ones_b, accrc.at[rb.at[k]], add=True)

    plsc.subcore_barrier()
    pltpu.sync_copy(accr.at[pl.ds(sid * zone, zone)],
                    rs_hbm.at[cid, pl.ds(sid * zone, zone)])
    pltpu.sync_copy(accrc.at[pl.ds(sid * zone, zone)], zc)
    pltpu.sync_copy(zc, rc_hbm.at[pl.ds(cid * RPAD + sid * zone, zone)])


_sc_res = pl.kernel(
    _sc_res_body,
    out_type=[jax.ShapeDtypeStruct((NC, RPAD, H), jnp.float32),
              jax.ShapeDtypeStruct((NC * RPAD,), jnp.float32)],
    mesh=plsc.VectorSubcoreMesh(core_axis_name="c", subcore_axis_name="s"),
    scratch_types=[
        pltpu.VMEM_SHARED((RPAD, H), jnp.float32),
        pltpu.VMEM_SHARED((RPAD,), jnp.float32),
        pltpu.VMEM((8, H), jnp.int32),
        pltpu.VMEM((H, H), jnp.float32),
        pltpu.VMEM((H,), jnp.float32),
        pltpu.VMEM((RPAD // NS,), jnp.float32),
        pltpu.SemaphoreType.DMA,
    ],
)


def kernel(x, edge_index, edge_attr, res_idx, W_enc, b_enc, eW1, eb1, eW2,
           eb2, mW1, mb1, mW2, mb2, gWih, gWhh, gbih, gbhh, lng, lnb, hW1,
           hb1, hW2, hb2):
    f32 = jnp.float32
    src = edge_index[0]
    dst = edge_index[1]

    npad_e = EPAD - NE
    pad_src = (jnp.arange(npad_e, dtype=jnp.int32) * 97) % NN
    pad_dst = NN + (jnp.arange(npad_e, dtype=jnp.int32) % (NPAD - NN))
    src2d = jnp.concatenate([src, pad_src]).reshape(IROWS, CHUNK)
    dst2d = jnp.concatenate([dst, pad_dst]).reshape(IROWS, CHUNK)
    ea_pad = jnp.concatenate(
        [edge_attr, jnp.zeros((npad_e, 16), f32)], axis=0)
    x_pad = jnp.concatenate([x, jnp.zeros((NPAD - NN, 128), f32)], axis=0)
    npad_r = NODE_PAD2 - NN
    pad_ridx = NRES + (jnp.arange(npad_r, dtype=jnp.int32) % (RPAD - NRES))
    ridx3d = jnp.concatenate([res_idx, pad_ridx]).reshape(NW, 3, H)
    ridx3d = jnp.pad(ridx3d, ((0, 0), (0, 5), (0, 0)),
                     constant_values=NRES)

    A = [mW1[l, :H, :] for l in range(3)]
    mb1_2 = [mb1[l].reshape(1, H) for l in range(3)]
    b_enc2 = b_enc.reshape(1, H)
    hb12 = hb1.reshape(1, 64)
    hb22 = hb2.reshape(1, 1)

    W2f, bf = _fold_weights(eW2, mW1, eb2)

    h, hp = _encode(x_pad, W_enc, b_enc2, A[0], mb1_2[0])

    eps = _edge_mlp(ea_pad, eW1, eb1, W2f, bf)

    for l in range(3):
        S, C = _sc_edge(hp, eps[l], src2d, dst2d)
        C = C.reshape(NC, NPAD)
        with_next = l < 2
        an = A[l + 1] if with_next else A[0]
        mb1n = mb1_2[l + 1] if with_next else mb1_2[0]
        h, hp = _layer_tail(
            S, C, h, mW2[l], mb2[l].reshape(1, H), gWih[l], gWhh[l],
            gbih[l].reshape(1, 3 * H), gbhh[l].reshape(1, 3 * H),
            lng[l].reshape(1, H), lnb[l].reshape(1, H), an, mb1n, with_next)

    h_rp = jnp.concatenate(
        [h, jnp.zeros((NODE_PAD2 - NPAD, H), f32)], axis=0)
    RS, RC = _sc_res(h_rp, ridx3d)
    out2d = _head(RS, RC.reshape(NC, RPAD), hW1, hb12, hW2, hb22)
    return out2d[:NRES, 0]
